# baseline (speedup 1.0000x reference)
import functools

import jax
import jax.numpy as jnp
from jax import lax
from jax.experimental import pallas as pl
from jax.experimental.pallas import tpu as pltpu
from jax.experimental.pallas import tpu_sc as plsc

N_NODES = 10000
N_EDGES = 320000
FEAT = 128
DIM_RBF = 16

_NUNIT = 4
_NE = N_EDGES // _NUNIT


_TE = 2000


def _msg_body(rbf_ref, m_ref, wt_ref, out_ref):
    e = jnp.dot(rbf_ref[...], wt_ref[...], preferred_element_type=jnp.float32)
    out_ref[...] = e * m_ref[...]


def _edge_messages(m, rbf, w_rbf_t, e0):
    b0 = e0 // _TE
    grid = (_NE // _TE,)
    return pl.pallas_call(
        _msg_body,
        grid=grid,
        in_specs=[
            pl.BlockSpec((_TE, DIM_RBF), lambda i: (i + b0, 0)),
            pl.BlockSpec((_TE, FEAT), lambda i: (i + b0, 0)),
            pl.BlockSpec((DIM_RBF, FEAT), lambda i: (0, 0)),
        ],
        out_specs=pl.BlockSpec((_TE, FEAT), lambda i: (i, 0)),
        out_shape=jax.ShapeDtypeStruct((_NE, FEAT), jnp.float32),
    )(rbf, m, w_rbf_t)


_NS = 16
_EPT = _NE // _NS
_CH = 128
_NCHF = _EPT // _CH
_TAIL = _EPT - _NCHF * _CH
_NPAD = 10240
_RPT = _NPAD // _NS


def _sc_scatter(msg, dst_unit, prev):
    mesh = plsc.VectorSubcoreMesh(core_axis_name="c", subcore_axis_name="s",
                                  num_cores=1)

    scratch = [
        pltpu.VMEM((2, _CH), jnp.int32),
        pltpu.VMEM((2, _CH, FEAT), jnp.float32),
        pltpu.VMEM_SHARED((_NPAD, FEAT), jnp.float32),
        pltpu.SemaphoreType.DMA,
        pltpu.SemaphoreType.DMA,
    ]
    if _TAIL:
        scratch += [pltpu.VMEM((1, _TAIL), jnp.int32),
                    pltpu.VMEM((_TAIL, FEAT), jnp.float32)]

    @functools.partial(
        pl.kernel,
        out_type=jax.ShapeDtypeStruct((_NPAD, FEAT), jnp.float32),
        mesh=mesh,
        scratch_types=scratch,
    )
    def scatter_kernel(msg_hbm, dst_hbm, prev_hbm, out_hbm, idx_v, msg_v,
                       acc_sh, sem0, sem1, *tail_bufs):
        s = lax.axis_index("s")
        base = s * _EPT
        sems = (sem0, sem1)
        r0 = s * _RPT

        pltpu.sync_copy(prev_hbm.at[pl.ds(r0, _RPT)],
                        acc_sh.at[pl.ds(r0, _RPT)])
        plsc.subcore_barrier()

        def start(j, b):
            pltpu.async_copy(dst_hbm.at[pl.ds(base + j * _CH, _CH)],
                             idx_v.at[b], sems[b])
            pltpu.async_copy(msg_hbm.at[pl.ds(base + j * _CH, _CH)],
                             msg_v.at[b], sems[b])

        def finish(j, b):
            pltpu.make_async_copy(dst_hbm.at[pl.ds(base + j * _CH, _CH)],
                                  idx_v.at[b], sems[b]).wait()
            pltpu.make_async_copy(msg_hbm.at[pl.ds(base + j * _CH, _CH)],
                                  msg_v.at[b], sems[b]).wait()
            pltpu.sync_copy(msg_v.at[b], acc_sh.at[idx_v.at[b]], add=True)

        start(0, 0)
        if _NCHF > 1:
            start(1, 1)

        def pair(i, carry):
            j0 = 2 * i
            finish(j0, 0)

            @pl.when(j0 + 2 < _NCHF)
            def _():
                start(j0 + 2, 0)

            finish(j0 + 1, 1)

            @pl.when(j0 + 3 < _NCHF)
            def _():
                start(j0 + 3, 1)

            return carry

        lax.fori_loop(0, (_NCHF - 1) // 2, pair, 0)
        if _NCHF % 2 == 1:
            finish(_NCHF - 1, 0)
        else:
            finish(_NCHF - 2, 0)
            finish(_NCHF - 1, 1)

        if _TAIL:
            idx_t, msg_t = tail_bufs
            toff = base + _NCHF * _CH
            pltpu.sync_copy(dst_hbm.at[pl.ds(toff, _TAIL)], idx_t.at[0])
            pltpu.sync_copy(msg_hbm.at[pl.ds(toff, _TAIL)], msg_t)
            pltpu.sync_copy(msg_t, acc_sh.at[idx_t.at[0]], add=True)

        plsc.subcore_barrier()

        pltpu.sync_copy(acc_sh.at[pl.ds(r0, _RPT)],
                        out_hbm.at[pl.ds(r0, _RPT)])

    return scatter_kernel(msg, dst_unit, prev)


_TN = 1000


def _silu(x):
    return x * jax.nn.sigmoid(x)


def _mlp_body(*refs):
    parts = refs[:-8]
    w1, b1, w2, b2, w3, b3, wo, out_ref = refs[-8:]
    x = parts[0][...]
    for p in parts[1:]:
        x = x + p[...]
    x = _silu(jnp.dot(x, w1[...], preferred_element_type=jnp.float32) + b1[...])
    x = _silu(jnp.dot(x, w2[...], preferred_element_type=jnp.float32) + b2[...])
    x = _silu(jnp.dot(x, w3[...], preferred_element_type=jnp.float32) + b3[...])
    out_ref[...] = jnp.dot(x, wo[...], preferred_element_type=jnp.float32)


def _mlp(parts, w1t, b1, w2t, b2, w3t, b3, wot):
    grid = (N_NODES // _TN,)
    full = lambda shape: pl.BlockSpec(shape, lambda i: tuple(0 for _ in shape))
    part_spec = pl.BlockSpec((_TN, FEAT), lambda i: (i, 0))
    return pl.pallas_call(
        _mlp_body,
        grid=grid,
        in_specs=[part_spec] * len(parts) + [
            full((FEAT, FEAT)), full((1, FEAT)),
            full((FEAT, FEAT)), full((1, FEAT)),
            full((FEAT, FEAT)), full((1, FEAT)),
            full((FEAT, 1)),
        ],
        out_specs=pl.BlockSpec((_TN, 1), lambda i: (i, 0)),
        out_shape=jax.ShapeDtypeStruct((N_NODES, 1), jnp.float32),
    )(*parts, w1t, b1, w2t, b2, w3t, b3, wot)


def kernel(m_ji, rbf_ji, atom_edge_index, W_rbf, W1, b1, W2, b2, W3, b3, W_out):
    dst = atom_edge_index[1].astype(jnp.int32)
    w_rbf_t = W_rbf.T
    zero_part = jnp.zeros((_NPAD, FEAT), jnp.float32)
    chains = [zero_part, zero_part]
    for u in range(_NUNIT):
        msg = _edge_messages(m_ji, rbf_ji, w_rbf_t, u * _NE)
        dst_unit = lax.slice_in_dim(dst, u * _NE, (u + 1) * _NE)
        h = u % 2
        chains[h] = _sc_scatter(msg, dst_unit, chains[h])
    return _mlp(chains,
                W1.T, b1[None, :], W2.T, b2[None, :], W3.T, b3[None, :],
                W_out.T)


# speedup vs baseline: 1.5791x; 1.5791x over previous
import functools

import jax
import jax.numpy as jnp
from jax import lax
from jax.experimental import pallas as pl
from jax.experimental.pallas import tpu as pltpu
from jax.experimental.pallas import tpu_sc as plsc

N_NODES = 10000
N_EDGES = 320000
FEAT = 128
DIM_RBF = 16

_NSLAB = 2
_NE = N_EDGES // _NSLAB


_TE = 3200


def _msg_body(rbft_ref, m_ref, wt_ref, out_ref):
    e = lax.dot_general(rbft_ref[...], wt_ref[...], (((0,), (0,)), ((), ())),
                        preferred_element_type=jnp.float32)
    out_ref[...] = e * m_ref[...]


def _edge_messages(m, rbf_t, w_rbf_t, e0):
    b0 = e0 // _TE
    grid = (_NE // _TE,)
    return pl.pallas_call(
        _msg_body,
        grid=grid,
        in_specs=[
            pl.BlockSpec((DIM_RBF, _TE), lambda i: (0, i + b0)),
            pl.BlockSpec((_TE, FEAT), lambda i: (i + b0, 0)),
            pl.BlockSpec((DIM_RBF, FEAT), lambda i: (0, 0)),
        ],
        out_specs=pl.BlockSpec((_TE, FEAT), lambda i: (i, 0)),
        out_shape=jax.ShapeDtypeStruct((_NE, FEAT), jnp.float32),
    )(rbf_t, m, w_rbf_t)


_NC = 2
_NS = 16
_NW = _NC * _NS
_EPT = _NE // _NW
_CH = 128
_NCHF = _EPT // _CH
_TAIL = _EPT - _NCHF * _CH
_NPAD = 10240
_RPT = _NPAD // _NS
_ZR = 128


def _sc_scatter(msg, dst_slab, prev):
    mesh = plsc.VectorSubcoreMesh(core_axis_name="c", subcore_axis_name="s")

    scratch = [
        pltpu.VMEM((2, _CH), jnp.int32),
        pltpu.VMEM((2, _CH, FEAT), jnp.float32),
        pltpu.VMEM_SHARED((_NPAD, FEAT), jnp.float32),
        pltpu.SemaphoreType.DMA,
        pltpu.SemaphoreType.DMA,
        pltpu.SemaphoreType.DMA,
        pltpu.SemaphoreType.DMA,
    ]
    if _TAIL:
        scratch += [pltpu.VMEM((1, _TAIL), jnp.int32),
                    pltpu.VMEM((_TAIL, FEAT), jnp.float32)]

    @functools.partial(
        pl.kernel,
        out_type=(jax.ShapeDtypeStruct((_NPAD, FEAT), jnp.float32),
                  jax.ShapeDtypeStruct((_NPAD, FEAT), jnp.float32)),
        mesh=mesh,
        scratch_types=scratch,
    )
    def scatter_kernel(msg_hbm, dst_hbm, prev0_hbm, prev1_hbm,
                       out0_hbm, out1_hbm, idx_v, msg_v,
                       acc_sh, sem0, sem1, ssem0, ssem1, *tail_bufs):
        c = lax.axis_index("c")
        s = lax.axis_index("s")
        tid = s * _NC + c
        mbase = tid * _EPT
        dbase = mbase
        sems = (sem0, sem1)
        r0 = s * _RPT

        @pl.when(c == 0)
        def _():
            pltpu.sync_copy(prev0_hbm.at[pl.ds(r0, _RPT)],
                            acc_sh.at[pl.ds(r0, _RPT)])

        @pl.when(c == 1)
        def _():
            pltpu.sync_copy(prev1_hbm.at[pl.ds(r0, _RPT)],
                            acc_sh.at[pl.ds(r0, _RPT)])

        plsc.subcore_barrier()

        ssems = (ssem0, ssem1)

        def start_in(j, b):
            pltpu.async_copy(dst_hbm.at[pl.ds(dbase + j * _CH, _CH)],
                             idx_v.at[b], sems[b])
            pltpu.async_copy(msg_hbm.at[pl.ds(mbase + j * _CH, _CH)],
                             msg_v.at[b], sems[b])

        def wait_in(j, b):
            pltpu.make_async_copy(dst_hbm.at[pl.ds(dbase + j * _CH, _CH)],
                                  idx_v.at[b], sems[b]).wait()
            pltpu.make_async_copy(msg_hbm.at[pl.ds(mbase + j * _CH, _CH)],
                                  msg_v.at[b], sems[b]).wait()

        def sc_start(b):
            pltpu.async_copy(msg_v.at[b], acc_sh.at[idx_v.at[b]], ssems[b],
                             add=True)

        def sc_wait(b):
            pltpu.make_async_copy(msg_v.at[b], acc_sh.at[idx_v.at[b]],
                                  ssems[b]).wait()

        start_in(0, 0)
        if _NCHF > 1:
            start_in(1, 1)
        wait_in(0, 0)
        sc_start(0)

        def step(j, b):
            wait_in(j, b)
            sc_start(b)
            sc_wait(1 - b)

            @pl.when(j + 1 < _NCHF)
            def _():
                start_in(j + 1, 1 - b)

        def pair(i, carry):
            step(2 * i + 1, 1)
            step(2 * i + 2, 0)
            return carry

        lax.fori_loop(0, (_NCHF - 1) // 2, pair, 0)
        if _NCHF % 2 == 0:
            step(_NCHF - 1, 1)
            sc_wait(1)
        else:
            sc_wait((_NCHF - 1) & 1)

        if _TAIL:
            idx_t, msg_t = tail_bufs
            toff_d = dbase + _NCHF * _CH
            toff_m = mbase + _NCHF * _CH
            pltpu.sync_copy(dst_hbm.at[pl.ds(toff_d, _TAIL)], idx_t.at[0])
            pltpu.sync_copy(msg_hbm.at[pl.ds(toff_m, _TAIL)], msg_t)
            pltpu.sync_copy(msg_t, acc_sh.at[idx_t.at[0]], add=True)

        plsc.subcore_barrier()

        @pl.when(c == 0)
        def _():
            pltpu.sync_copy(acc_sh.at[pl.ds(r0, _RPT)],
                            out0_hbm.at[pl.ds(r0, _RPT)])

        @pl.when(c == 1)
        def _():
            pltpu.sync_copy(acc_sh.at[pl.ds(r0, _RPT)],
                            out1_hbm.at[pl.ds(r0, _RPT)])

    return scatter_kernel(msg, dst_slab, prev[0], prev[1])


_TN = 1024


def _silu(x):
    return x * jax.nn.sigmoid(x)


def _mlp_body(*refs):
    parts = refs[:-8]
    w1, b1, w2, b2, w3, b3, wo, out_ref = refs[-8:]
    x = parts[0][...]
    for p in parts[1:]:
        x = x + p[...]
    x = _silu(jnp.dot(x, w1[...], preferred_element_type=jnp.float32) + b1[...])
    x = _silu(jnp.dot(x, w2[...], preferred_element_type=jnp.float32) + b2[...])
    x = _silu(jnp.dot(x, w3[...], preferred_element_type=jnp.float32) + b3[...])
    out_ref[...] = lax.dot_general(wo[...], x, (((1,), (1,)), ((), ())),
                                   preferred_element_type=jnp.float32)


def _mlp(parts, w1t, b1, w2t, b2, w3t, b3, wot):
    grid = (_NPAD // _TN,)
    full = lambda shape: pl.BlockSpec(shape, lambda i: tuple(0 for _ in shape))
    part_spec = pl.BlockSpec((_TN, FEAT), lambda i: (i, 0))
    return pl.pallas_call(
        _mlp_body,
        grid=grid,
        in_specs=[part_spec] * len(parts) + [
            full((FEAT, FEAT)), full((1, FEAT)),
            full((FEAT, FEAT)), full((1, FEAT)),
            full((FEAT, FEAT)), full((1, FEAT)),
            full((1, FEAT)),
        ],
        out_specs=pl.BlockSpec((1, _TN), lambda i: (0, i)),
        out_shape=jax.ShapeDtypeStruct((1, _NPAD), jnp.float32),
    )(*parts, w1t, b1, w2t, b2, w3t, b3, wot)


def kernel(m_ji, rbf_ji, atom_edge_index, W_rbf, W1, b1, W2, b2, W3, b3, W_out):
    dst = atom_edge_index[1].astype(jnp.int32)
    rbf_t = rbf_ji.T
    w_rbf_t = W_rbf.T
    zero_part = jnp.zeros((_NPAD, FEAT), jnp.float32)
    parts = (zero_part, zero_part)
    for sl in range(_NSLAB):
        msg = _edge_messages(m_ji, rbf_t, w_rbf_t, sl * _NE)
        dst_slab = lax.slice_in_dim(dst, sl * _NE, (sl + 1) * _NE)
        parts = _sc_scatter(msg, dst_slab, parts)
    out_t = _mlp(list(parts),
                 W1.T, b1[None, :], W2.T, b2[None, :], W3.T, b3[None, :],
                 W_out)
    return out_t[:, :N_NODES].T
